# baseline (speedup 1.0000x reference)
import functools
import math

import jax
import jax.numpy as jnp
from jax import lax
from jax.experimental import pallas as pl
from jax.experimental.pallas import tpu as pltpu
from jax.experimental.pallas import tpu_sc as plsc

D_MODEL = 64
SCALE = math.sqrt(D_MODEL)

_INFO = plsc.get_sparse_core_info()
NUM_WORKERS = _INFO.num_cores * _INFO.num_subcores

SBLK = 128
NBUF = 2
T_LEN = 200
N_SEQ = 16384
NSJ = N_SEQ // SBLK
SJ_PER_TILE = NSJ // NUM_WORKERS


def _emb_kernel(xt_hbm, table_hbm, z_hbm, idx_v, rows_v, zbuf_v, ktab_v,
                *sems):
    gsem = sems[0:NBUF]
    osem = sems[NBUF:2 * NBUF]
    wid = lax.axis_index("s") * _INFO.num_cores + lax.axis_index("c")

    iota = lax.iota(jnp.int32, 16)
    for b in range(NBUF):
        for k in range(4):
            ktab_v[4 * b + k, pl.ds(0, 16)] = (
                b * 8192 + (2 * k + iota // 8) * 1024 + (iota % 8) * 128)

    def gather_start(b):
        pltpu.async_copy(table_hbm.at[idx_v.at[b]], rows_v.at[b], gsem[b])

    def gather_wait(b):
        pltpu.make_async_copy(table_hbm.at[idx_v.at[b]],
                              rows_v.at[b], gsem[b]).wait()

    def zstore_start(t, sj, b):
        for dhi in range(8):
            pltpu.async_copy(zbuf_v.at[pl.ds(b * 8192 + dhi * 1024, 1024)],
                             z_hbm.at[t, dhi, sj], osem[b])

    def zstore_wait(b):
        for _ in range(8):
            pltpu.make_async_copy(zbuf_v.at[pl.ds(0, 1024)],
                                  z_hbm.at[0, 0, 0], osem[b]).wait()

    def transpose_scale(b):
        for k in range(4):
            ktab_v[8 + 4 * b + k, pl.ds(0, 16)] = (
                ktab_v[4 * b + k, pl.ds(0, 16)] * 1)

        def sb(sl, _):
            for k in range(4):
                v = rows_v[b, sl, pl.ds(k * 16, 16)] * SCALE
                idx = ktab_v[8 + 4 * b + k, pl.ds(0, 16)]
                plsc.store_scatter(zbuf_v, [idx], v)
                ktab_v[8 + 4 * b + k, pl.ds(0, 16)] = idx + 1
            return ()

        lax.fori_loop(0, SBLK, sb, (), unroll=2)

    for sjo in range(SJ_PER_TILE):
        sj = wid * SJ_PER_TILE + sjo
        s0 = sj * SBLK

        pltpu.sync_copy(xt_hbm.at[0, pl.ds(s0, SBLK)], idx_v.at[0])
        gather_start(0)

        def pair(p, _):
            for b in range(2):
                t = 2 * p + b

                @pl.when(t + 1 < T_LEN)
                def _():
                    pltpu.sync_copy(xt_hbm.at[t + 1, pl.ds(s0, SBLK)],
                                    idx_v.at[1 - b])
                    gather_start(1 - b)

                gather_wait(b)

                @pl.when(t >= 2)
                def _():
                    zstore_wait(b)

                transpose_scale(b)
                zstore_start(t, sj, b)
            return ()

        lax.fori_loop(0, T_LEN // 2, pair, ())
        zstore_wait(0)
        zstore_wait(1)


def kernel(x, table):
    mesh = plsc.VectorSubcoreMesh(core_axis_name="c", subcore_axis_name="s")
    run = pl.kernel(
        _emb_kernel,
        out_type=jax.ShapeDtypeStruct((T_LEN, 8, NSJ, 1024), jnp.float32),
        mesh=mesh,
        scratch_types=(
            [pltpu.VMEM((NBUF, SBLK), jnp.int32),
             pltpu.VMEM((NBUF, SBLK, D_MODEL), jnp.float32),
             pltpu.VMEM((NBUF * 8192,), jnp.float32),
             pltpu.VMEM((16, 16), jnp.int32)]
            + [pltpu.SemaphoreType.DMA] * (2 * NBUF)
        ),
        compiler_params=pltpu.CompilerParams(use_tc_tiling_on_sc=False, needs_layout_passes=False),
    )
    z = run(x.T, table)
    z5 = z.reshape(T_LEN, 8, NSJ, 8, SBLK)
    return z5.transpose(2, 4, 0, 1, 3).reshape(N_SEQ, T_LEN, D_MODEL)


# speedup vs baseline: 1.0369x; 1.0369x over previous
import functools
import math

import jax
import jax.numpy as jnp
from jax import lax
from jax.experimental import pallas as pl
from jax.experimental.pallas import tpu as pltpu
from jax.experimental.pallas import tpu_sc as plsc

D_MODEL = 64
SCALE = math.sqrt(D_MODEL)

_INFO = plsc.get_sparse_core_info()
NUM_WORKERS = _INFO.num_cores * _INFO.num_subcores

SBLK = 128
NBUF = 2
T_LEN = 200
N_SEQ = 16384
NSJ = N_SEQ // SBLK
SJ_PER_TILE = NSJ // NUM_WORKERS


def _emb_kernel(xt_hbm, table_hbm, z_hbm, idx_v, rows_v, zbuf_v, ktab_v,
                *sems):
    gsem = sems[0:NBUF]
    osem = sems[NBUF:2 * NBUF]
    wid = lax.axis_index("s") * _INFO.num_cores + lax.axis_index("c")

    iota = lax.iota(jnp.int32, 16)
    for b in range(NBUF):
        for k in range(4):
            ktab_v[4 * b + k, pl.ds(0, 16)] = (
                b * 8192 + (2 * k + iota // 8) * 1024 + (iota % 8) * 128)

    def gather_start(b):
        pltpu.async_copy(table_hbm.at[idx_v.at[b]], rows_v.at[b], gsem[b])

    def gather_wait(b):
        pltpu.make_async_copy(table_hbm.at[idx_v.at[b]],
                              rows_v.at[b], gsem[b]).wait()

    def zstore_start(t, sj, b):
        for dhi in range(8):
            pltpu.async_copy(zbuf_v.at[pl.ds(b * 8192 + dhi * 1024, 1024)],
                             z_hbm.at[t, dhi, sj], osem[b])

    def zstore_wait(b):
        for _ in range(8):
            pltpu.make_async_copy(zbuf_v.at[pl.ds(0, 1024)],
                                  z_hbm.at[0, 0, 0], osem[b]).wait()

    def transpose_scale(b):
        init = tuple(ktab_v[4 * b + k, pl.ds(0, 16)] for k in range(4))

        def sb(sl, idxs):
            for k in range(4):
                v = rows_v[b, sl, pl.ds(k * 16, 16)] * SCALE
                plsc.store_scatter(zbuf_v, [idxs[k]], v)
            return tuple(i + 1 for i in idxs)

        lax.fori_loop(0, SBLK, sb, init, unroll=4)

    for sjo in range(SJ_PER_TILE):
        sj = wid * SJ_PER_TILE + sjo
        s0 = sj * SBLK

        pltpu.sync_copy(xt_hbm.at[0, pl.ds(s0, SBLK)], idx_v.at[0])
        gather_start(0)

        def pair(p, _):
            for b in range(2):
                t = 2 * p + b

                @pl.when(t + 1 < T_LEN)
                def _():
                    pltpu.sync_copy(xt_hbm.at[t + 1, pl.ds(s0, SBLK)],
                                    idx_v.at[1 - b])
                    gather_start(1 - b)

                gather_wait(b)

                @pl.when(t >= 2)
                def _():
                    zstore_wait(b)

                transpose_scale(b)
                zstore_start(t, sj, b)
            return ()

        lax.fori_loop(0, T_LEN // 2, pair, ())
        zstore_wait(0)
        zstore_wait(1)


def kernel(x, table):
    mesh = plsc.VectorSubcoreMesh(core_axis_name="c", subcore_axis_name="s")
    run = pl.kernel(
        _emb_kernel,
        out_type=jax.ShapeDtypeStruct((T_LEN, 8, NSJ, 1024), jnp.float32),
        mesh=mesh,
        scratch_types=(
            [pltpu.VMEM((NBUF, SBLK), jnp.int32),
             pltpu.VMEM((NBUF, SBLK, D_MODEL), jnp.float32),
             pltpu.VMEM((NBUF * 8192,), jnp.float32),
             pltpu.VMEM((16, 16), jnp.int32)]
            + [pltpu.SemaphoreType.DMA] * (2 * NBUF)
        ),
        compiler_params=pltpu.CompilerParams(use_tc_tiling_on_sc=False, needs_layout_passes=False),
    )
    z = run(x.T, table)
    z5 = z.reshape(T_LEN, 8, NSJ, 8, SBLK)
    return z5.transpose(2, 4, 0, 1, 3).reshape(N_SEQ, T_LEN, D_MODEL)


# speedup vs baseline: 2.9945x; 2.8878x over previous
import functools
import math

import jax
import jax.numpy as jnp
from jax import lax
from jax.experimental import pallas as pl
from jax.experimental.pallas import tpu as pltpu
from jax.experimental.pallas import tpu_sc as plsc

D_MODEL = 64
SCALE = math.sqrt(D_MODEL)

_INFO = plsc.get_sparse_core_info()
NUM_WORKERS = _INFO.num_cores * _INFO.num_subcores

NBUF = 4


def _emb_kernel(n_seq, seq_len, x_hbm, table_hbm, out_hbm, idx_v, rows_v,
                *sems):
    gsem = sems[0:NBUF]
    isem = sems[NBUF:2 * NBUF]
    osem = sems[2 * NBUF:3 * NBUF]
    wid = lax.axis_index("s") * _INFO.num_cores + lax.axis_index("c")
    per_tile = n_seq // NUM_WORKERS
    base = wid * per_tile

    def idx_start(c, b):
        pltpu.async_copy(x_hbm.at[base + c], idx_v.at[b], isem[b])

    def idx_wait(b):
        pltpu.make_async_copy(x_hbm.at[base], idx_v.at[b], isem[b]).wait()

    def gather_start(b):
        pltpu.async_copy(table_hbm.at[idx_v.at[b]], rows_v.at[b], gsem[b])

    def gather_wait(b):
        pltpu.make_async_copy(table_hbm.at[idx_v.at[b]],
                              rows_v.at[b], gsem[b]).wait()

    def ostore_start(c, b):
        pltpu.async_copy(
            rows_v.at[b],
            out_hbm.at[pl.ds((base + c) * seq_len, seq_len), pl.ds(0, D_MODEL)],
            osem[b])

    def ostore_wait(b):
        pltpu.make_async_copy(
            rows_v.at[b],
            out_hbm.at[pl.ds(base * seq_len, seq_len), pl.ds(0, D_MODEL)],
            osem[b]).wait()

    def scale(b):
        def sb(j, _):
            for k in range(D_MODEL // 16):
                rows_v[b, j, pl.ds(k * 16, 16)] = (
                    rows_v[b, j, pl.ds(k * 16, 16)] * SCALE)
            return ()

        lax.fori_loop(0, seq_len, sb, (), unroll=8)

    def do_chunk(i, b, launch_gather, wait_ostore, launch_idx):
        bj = (b + 2) % NBUF
        if launch_gather:
            if wait_ostore:
                ostore_wait(bj)
            idx_wait(bj)
            gather_start(bj)
        gather_wait(b)
        if launch_idx:
            idx_start(i + 4, b)
        scale(b)
        ostore_start(i, b)

    pltpu.sync_copy(x_hbm.at[base], idx_v.at[0])
    gather_start(0)
    pltpu.sync_copy(x_hbm.at[base + 1], idx_v.at[1])
    gather_start(1)
    idx_start(2, 2)
    idx_start(3, 3)

    do_chunk(0, 0, True, False, True)
    do_chunk(1, 1, True, False, True)
    do_chunk(2, 2, True, True, True)
    do_chunk(3, 3, True, True, True)

    steps = per_tile // NBUF

    def body(s, _):
        i0 = s * NBUF
        for b in range(NBUF):
            do_chunk(i0 + b, b, True, True, True)
        return ()

    lax.fori_loop(1, steps - 1, body, ())

    n = per_tile
    do_chunk(n - 4, 0, True, True, False)
    do_chunk(n - 3, 1, True, True, False)
    do_chunk(n - 2, 2, False, False, False)
    do_chunk(n - 1, 3, False, False, False)

    for b in range(NBUF):
        ostore_wait(b)


def kernel(x, table):
    n_seq, seq_len = x.shape
    assert n_seq % (NUM_WORKERS * NBUF) == 0

    mesh = plsc.VectorSubcoreMesh(core_axis_name="c", subcore_axis_name="s")
    run = pl.kernel(
        functools.partial(_emb_kernel, n_seq, seq_len),
        out_type=jax.ShapeDtypeStruct((n_seq * seq_len, 128), jnp.float32),
        mesh=mesh,
        scratch_types=(
            [pltpu.VMEM((NBUF, seq_len), jnp.int32),
             pltpu.VMEM((NBUF, seq_len, D_MODEL), jnp.float32)]
            + [pltpu.SemaphoreType.DMA] * (3 * NBUF)
        ),
        compiler_params=pltpu.CompilerParams(use_tc_tiling_on_sc=False),
    )
    out128 = run(x, table)
    return out128[:, :D_MODEL].reshape(n_seq, seq_len, D_MODEL)
